# baseline (speedup 1.0000x reference)
import functools

import jax
import jax.numpy as jnp
from jax import lax
from jax.experimental import pallas as pl
from jax.experimental.pallas import tpu as pltpu
from jax.experimental.pallas import tpu_sc as plsc

_K_HIGH = 16
_K_LOW = 2


def _topk_body(sig_ref, x_ref, idx_ref, deg_ref, sim_ref, *, bt: int, t: int,
               zero_idx: int):
    b = pl.program_id(0)
    i = pl.program_id(1)
    xk = x_ref[0]
    q = x_ref[0, pl.ds(i * bt, bt), :]

    kn = xk / jnp.clip(jnp.sqrt(jnp.sum(xk * xk, axis=1, keepdims=True)),
                       1e-12, None)
    qn = q / jnp.clip(jnp.sqrt(jnp.sum(q * q, axis=1, keepdims=True)),
                      1e-12, None)

    sim = jax.lax.dot_general(qn, kn, (((1,), (1,)), ((), ())),
                              preferred_element_type=jnp.float32)

    iota_s = jax.lax.broadcasted_iota(jnp.int32, (bt, t), 1)
    t_glob = i * bt + jax.lax.broadcasted_iota(jnp.int32, (bt, t), 0)
    sim_ref[...] = jnp.where(iota_s < t_glob, sim, jnp.float32(-1e9))

    sigma = sig_ref[0, 0]
    surp = jnp.tanh(sigma * jnp.mean(jnp.abs(q), axis=1, keepdims=True))
    kt = jnp.clip(jnp.round(_K_LOW + (_K_HIGH - _K_LOW) * surp),
                  0.0, float(min(_K_HIGH, t - 1)))

    vals, idxs = [], []
    for j in range(_K_HIGH):
        s = sim_ref[...]
        cur = jnp.max(s, axis=1, keepdims=True)
        idx = jnp.argmax(s, axis=1).reshape(bt, 1)
        sim_ref[...] = jnp.where(iota_s == idx, jnp.float32(-2e9), s)
        vals.append(cur)
        idxs.append(idx)
    v16 = jnp.concatenate(vals, axis=1)
    i16 = jnp.concatenate(idxs, axis=1)

    jj = jax.lax.broadcasted_iota(jnp.int32, (bt, _K_HIGH), 1)
    kti = kt.astype(jnp.int32)
    sel = jnp.logical_and(jj < kti, v16 > -1e8)
    gidx = jnp.where(sel, i16 + b * t, jnp.int32(zero_idx))
    idx_ref[0] = gidx
    deg_ref[0] = jnp.maximum(jnp.sum(sel.astype(jnp.float32), axis=1,
                                     keepdims=True), 1.0)


def _sc_gather_sum(xtab, idxf, *, n_tok: int, d: int):
    info = plsc.get_sparse_core_info()
    nw = info.num_cores * info.num_subcores
    tok_w = n_tok // nw
    mesh = plsc.VectorSubcoreMesh(core_axis_name="c", subcore_axis_name="s")

    gt = 2
    rows_g = gt * _K_HIGH
    ng = tok_w // gt

    @functools.partial(
        pl.kernel, mesh=mesh,
        out_type=jax.ShapeDtypeStruct((n_tok, d), jnp.float32),
        scratch_types=[
            pltpu.VMEM((tok_w * _K_HIGH,), jnp.int32),
            pltpu.VMEM((rows_g, d), jnp.float32),
            pltpu.VMEM((rows_g, d), jnp.float32),
            pltpu.VMEM((gt, d), jnp.float32),
            pltpu.SemaphoreType.DMA,
            pltpu.SemaphoreType.DMA,
        ],
    )
    def k(xtab_hbm, idx_hbm, out_hbm, idx_v, buf0, buf1, msg_v, sem0, sem1):
        wid = lax.axis_index("s") * info.num_cores + lax.axis_index("c")
        base = wid * tok_w
        pltpu.sync_copy(idx_hbm.at[pl.ds(base * _K_HIGH, tok_w * _K_HIGH)],
                        idx_v)

        bufs = (buf0, buf1)
        sems = (sem0, sem1)
        pltpu.make_async_copy(
            xtab_hbm.at[idx_v.at[pl.ds(0, rows_g)]], buf0, sem0).start()

        def step(g2, _):
            for par in range(2):
                cur_buf, cur_sem = bufs[par], sems[par]
                nxt_buf, nxt_sem = bufs[1 - par], sems[1 - par]
                g = g2 + par
                pltpu.make_async_copy(
                    xtab_hbm.at[idx_v.at[pl.ds(g * rows_g, rows_g)]],
                    cur_buf, cur_sem).wait()
                @pl.when(g + 1 < ng)
                def _():
                    pltpu.make_async_copy(
                        xtab_hbm.at[idx_v.at[pl.ds((g + 1) * rows_g,
                                                   rows_g)]],
                        nxt_buf, nxt_sem).start()

                def col(cc, _):
                    for tkk in range(gt):
                        acc = cur_buf[tkk * _K_HIGH, pl.ds(cc * 16, 16)]
                        for r in range(1, _K_HIGH):
                            acc = acc + cur_buf[tkk * _K_HIGH + r,
                                                pl.ds(cc * 16, 16)]
                        msg_v[tkk, pl.ds(cc * 16, 16)] = acc
                    return 0
                lax.fori_loop(0, d // 16, col, 0, unroll=2)
                pltpu.sync_copy(msg_v, out_hbm.at[pl.ds(base + g * gt, gt)])
            return 0
        lax.fori_loop(0, ng // 2, lambda h, c: step(2 * h, c), 0)

    return k(xtab, idxf.reshape(n_tok * _K_HIGH))


def _blend_body(mix_ref, scl_ref, x_ref, m_ref, d_ref, gain_ref, bias_ref,
                out_ref, *, bt: int, t: int):
    x = x_ref[0]
    msum = m_ref[0]
    deg = d_ref[0]

    mix = mix_ref[0, 0]
    scale = scl_ref[0, 0]
    blended = mix * x + (1.0 - mix) * (msum / deg)
    y = blended * gain_ref[0] + bias_ref[0]
    gelu = 0.5 * y * (1.0 + jax.lax.erf(y * jnp.float32(0.7071067811865476)))
    out_ref[0] = gelu * scale


def _stage_a(x, sigma, *, bt: int, interpret: bool = False):
    b, t, d = x.shape
    return pl.pallas_call(
        functools.partial(_topk_body, bt=bt, t=t, zero_idx=b * t),
        grid=(b, t // bt),
        in_specs=[
            pl.BlockSpec((1, 1), lambda bb, ii: (0, 0),
                         memory_space=pltpu.SMEM),
            pl.BlockSpec((1, t, d), lambda bb, ii: (bb, 0, 0)),
        ],
        out_specs=[
            pl.BlockSpec((1, bt, _K_HIGH), lambda bb, ii: (bb, ii, 0)),
            pl.BlockSpec((1, bt, 1), lambda bb, ii: (bb, ii, 0)),
        ],
        out_shape=[
            jax.ShapeDtypeStruct((b, t, _K_HIGH), jnp.int32),
            jax.ShapeDtypeStruct((b, t, 1), jnp.float32),
        ],
        scratch_shapes=[pltpu.VMEM((bt, t), jnp.float32)],
        interpret=interpret,
    )(sigma, x)


def _stage_c(x, msum, deg, mix, scale, gain, bias, *, bt: int,
             interpret: bool = False):
    b, t, d = x.shape
    return pl.pallas_call(
        functools.partial(_blend_body, bt=bt, t=t),
        grid=(b, t // bt),
        in_specs=[
            pl.BlockSpec((1, 1), lambda bb, ii: (0, 0),
                         memory_space=pltpu.SMEM),
            pl.BlockSpec((1, 1), lambda bb, ii: (0, 0),
                         memory_space=pltpu.SMEM),
            pl.BlockSpec((1, bt, d), lambda bb, ii: (bb, ii, 0)),
            pl.BlockSpec((1, bt, d), lambda bb, ii: (bb, ii, 0)),
            pl.BlockSpec((1, bt, 1), lambda bb, ii: (bb, ii, 0)),
            pl.BlockSpec((1, d), lambda bb, ii: (0, 0)),
            pl.BlockSpec((1, d), lambda bb, ii: (0, 0)),
        ],
        out_specs=pl.BlockSpec((1, bt, d), lambda bb, ii: (bb, ii, 0)),
        out_shape=jax.ShapeDtypeStruct((b, t, d), jnp.float32),
        interpret=interpret,
    )(mix, scale, x, msum, deg, gain, bias)


@jax.jit
def kernel(x, gain, bias, log_sigma_raw, log_mix, log_scale):
    b, t, d = x.shape
    bt = 256

    sigma = (jax.nn.softplus(log_sigma_raw) + 0.01).reshape(1, 1)
    mix = jax.nn.sigmoid(log_mix).reshape(1, 1)
    scale = (jax.nn.softplus(log_scale) + 0.01).reshape(1, 1)
    sigma = sigma.astype(jnp.float32)

    idx, deg = _stage_a(x, sigma, bt=bt)
    xtab = jnp.concatenate(
        [x.reshape(b * t, d), jnp.zeros((8, d), jnp.float32)], axis=0)
    msum = _sc_gather_sum(xtab, idx.reshape(b * t, _K_HIGH),
                          n_tok=b * t, d=d)
    return _stage_c(x, msum.reshape(b, t, d), deg,
                    mix.astype(jnp.float32), scale.astype(jnp.float32),
                    gain.reshape(1, d), bias.reshape(1, d), bt=bt)


# speedup vs baseline: 1.0017x; 1.0017x over previous
import functools

import jax
import jax.numpy as jnp
from jax import lax
from jax.experimental import pallas as pl
from jax.experimental.pallas import tpu as pltpu
from jax.experimental.pallas import tpu_sc as plsc

_K_HIGH = 16
_K_LOW = 2


def _topk_body(sig_ref, x_ref, idx_ref, deg_ref, sim_ref, *, bt: int, t: int,
               zero_idx: int):
    b = pl.program_id(0)
    i = pl.program_id(1)
    xk = x_ref[0]
    q = x_ref[0, pl.ds(i * bt, bt), :]

    kn = xk / jnp.clip(jnp.sqrt(jnp.sum(xk * xk, axis=1, keepdims=True)),
                       1e-12, None)
    qn = q / jnp.clip(jnp.sqrt(jnp.sum(q * q, axis=1, keepdims=True)),
                      1e-12, None)

    sim = jax.lax.dot_general(qn, kn, (((1,), (1,)), ((), ())),
                              preferred_element_type=jnp.float32)

    iota_s = jax.lax.broadcasted_iota(jnp.int32, (bt, t), 1)
    t_glob = i * bt + jax.lax.broadcasted_iota(jnp.int32, (bt, t), 0)
    sim_ref[...] = jnp.where(iota_s < t_glob, sim, jnp.float32(-1e9))

    sigma = sig_ref[0, 0]
    surp = jnp.tanh(sigma * jnp.mean(jnp.abs(q), axis=1, keepdims=True))
    kt = jnp.clip(jnp.round(_K_LOW + (_K_HIGH - _K_LOW) * surp),
                  0.0, float(min(_K_HIGH, t - 1)))

    vals, idxs = [], []
    for j in range(_K_HIGH):
        s = sim_ref[...]
        cur = jnp.max(s, axis=1, keepdims=True)
        idx = jnp.argmax(s, axis=1).reshape(bt, 1)
        sim_ref[...] = jnp.where(iota_s == idx, jnp.float32(-2e9), s)
        vals.append(cur)
        idxs.append(idx)
    v16 = jnp.concatenate(vals, axis=1)
    i16 = jnp.concatenate(idxs, axis=1)

    jj = jax.lax.broadcasted_iota(jnp.int32, (bt, _K_HIGH), 1)
    kti = kt.astype(jnp.int32)
    sel = jnp.logical_and(jj < kti, v16 > -1e8)
    gidx = jnp.where(sel, i16 + b * t, jnp.int32(zero_idx))
    idx_ref[0] = gidx
    deg_ref[0] = jnp.maximum(jnp.sum(sel.astype(jnp.float32), axis=1,
                                     keepdims=True), 1.0)


def _sc_gather_sum(xtab, idxf, *, n_tok: int, d: int):
    info = plsc.get_sparse_core_info()
    nw = info.num_cores * info.num_subcores
    tok_w = n_tok // nw
    mesh = plsc.VectorSubcoreMesh(core_axis_name="c", subcore_axis_name="s")

    gt = 2
    rows_g = gt * _K_HIGH
    ng = tok_w // gt

    @functools.partial(
        pl.kernel, mesh=mesh,
        out_type=jax.ShapeDtypeStruct((n_tok, d), jnp.float32),
        scratch_types=[
            pltpu.VMEM((tok_w * _K_HIGH,), jnp.int32),
            pltpu.VMEM((rows_g, d), jnp.float32),
            pltpu.VMEM((rows_g, d), jnp.float32),
            pltpu.VMEM((gt, d), jnp.float32),
            pltpu.SemaphoreType.DMA,
            pltpu.SemaphoreType.DMA,
        ],
    )
    def k(xtab_hbm, idx_hbm, out_hbm, idx_v, buf0, buf1, msg_v, sem0, sem1):
        wid = lax.axis_index("s") * info.num_cores + lax.axis_index("c")
        base = wid * tok_w
        pltpu.sync_copy(idx_hbm.at[pl.ds(base * _K_HIGH, tok_w * _K_HIGH)],
                        idx_v)

        bufs = (buf0, buf1)
        sems = (sem0, sem1)
        pltpu.make_async_copy(
            xtab_hbm.at[idx_v.at[pl.ds(0, rows_g)]], buf0, sem0).start()

        def step(g2, _):
            for par in range(2):
                cur_buf, cur_sem = bufs[par], sems[par]
                nxt_buf, nxt_sem = bufs[1 - par], sems[1 - par]
                g = g2 + par
                pltpu.make_async_copy(
                    xtab_hbm.at[idx_v.at[pl.ds(g * rows_g, rows_g)]],
                    cur_buf, cur_sem).wait()
                @pl.when(g + 1 < ng)
                def _():
                    pltpu.make_async_copy(
                        xtab_hbm.at[idx_v.at[pl.ds((g + 1) * rows_g,
                                                   rows_g)]],
                        nxt_buf, nxt_sem).start()

                def col(cc, _):
                    for tkk in range(gt):
                        vs = [cur_buf[tkk * _K_HIGH + r, pl.ds(cc * 16, 16)]
                              for r in range(_K_HIGH)]
                        while len(vs) > 1:
                            vs = [vs[z] + vs[z + 1]
                                  for z in range(0, len(vs), 2)]
                        msg_v[tkk, pl.ds(cc * 16, 16)] = vs[0]
                    return 0
                lax.fori_loop(0, d // 16, col, 0, unroll=4)
                pltpu.sync_copy(msg_v, out_hbm.at[pl.ds(base + g * gt, gt)])
            return 0
        lax.fori_loop(0, ng // 2, lambda h, c: step(2 * h, c), 0)

    return k(xtab, idxf.reshape(n_tok * _K_HIGH))


def _blend_body(mix_ref, scl_ref, x_ref, m_ref, d_ref, gain_ref, bias_ref,
                out_ref, *, bt: int, t: int):
    x = x_ref[0]
    msum = m_ref[0]
    deg = d_ref[0]

    mix = mix_ref[0, 0]
    scale = scl_ref[0, 0]
    blended = mix * x + (1.0 - mix) * (msum / deg)
    y = blended * gain_ref[0] + bias_ref[0]
    gelu = 0.5 * y * (1.0 + jax.lax.erf(y * jnp.float32(0.7071067811865476)))
    out_ref[0] = gelu * scale


def _stage_a(x, sigma, *, bt: int, interpret: bool = False):
    b, t, d = x.shape
    return pl.pallas_call(
        functools.partial(_topk_body, bt=bt, t=t, zero_idx=b * t),
        grid=(b, t // bt),
        in_specs=[
            pl.BlockSpec((1, 1), lambda bb, ii: (0, 0),
                         memory_space=pltpu.SMEM),
            pl.BlockSpec((1, t, d), lambda bb, ii: (bb, 0, 0)),
        ],
        out_specs=[
            pl.BlockSpec((1, bt, _K_HIGH), lambda bb, ii: (bb, ii, 0)),
            pl.BlockSpec((1, bt, 1), lambda bb, ii: (bb, ii, 0)),
        ],
        out_shape=[
            jax.ShapeDtypeStruct((b, t, _K_HIGH), jnp.int32),
            jax.ShapeDtypeStruct((b, t, 1), jnp.float32),
        ],
        scratch_shapes=[pltpu.VMEM((bt, t), jnp.float32)],
        interpret=interpret,
    )(sigma, x)


def _stage_c(x, msum, deg, mix, scale, gain, bias, *, bt: int,
             interpret: bool = False):
    b, t, d = x.shape
    return pl.pallas_call(
        functools.partial(_blend_body, bt=bt, t=t),
        grid=(b, t // bt),
        in_specs=[
            pl.BlockSpec((1, 1), lambda bb, ii: (0, 0),
                         memory_space=pltpu.SMEM),
            pl.BlockSpec((1, 1), lambda bb, ii: (0, 0),
                         memory_space=pltpu.SMEM),
            pl.BlockSpec((1, bt, d), lambda bb, ii: (bb, ii, 0)),
            pl.BlockSpec((1, bt, d), lambda bb, ii: (bb, ii, 0)),
            pl.BlockSpec((1, bt, 1), lambda bb, ii: (bb, ii, 0)),
            pl.BlockSpec((1, d), lambda bb, ii: (0, 0)),
            pl.BlockSpec((1, d), lambda bb, ii: (0, 0)),
        ],
        out_specs=pl.BlockSpec((1, bt, d), lambda bb, ii: (bb, ii, 0)),
        out_shape=jax.ShapeDtypeStruct((b, t, d), jnp.float32),
        interpret=interpret,
    )(mix, scale, x, msum, deg, gain, bias)


@jax.jit
def kernel(x, gain, bias, log_sigma_raw, log_mix, log_scale):
    b, t, d = x.shape
    bt = 256

    sigma = (jax.nn.softplus(log_sigma_raw) + 0.01).reshape(1, 1)
    mix = jax.nn.sigmoid(log_mix).reshape(1, 1)
    scale = (jax.nn.softplus(log_scale) + 0.01).reshape(1, 1)
    sigma = sigma.astype(jnp.float32)

    idx, deg = _stage_a(x, sigma, bt=bt)
    xtab = jnp.concatenate(
        [x.reshape(b * t, d), jnp.zeros((8, d), jnp.float32)], axis=0)
    msum = _sc_gather_sum(xtab, idx.reshape(b * t, _K_HIGH),
                          n_tok=b * t, d=d)
    return _stage_c(x, msum.reshape(b, t, d), deg,
                    mix.astype(jnp.float32), scale.astype(jnp.float32),
                    gain.reshape(1, d), bias.reshape(1, d), bt=bt)


# speedup vs baseline: 6.6956x; 6.6840x over previous
import functools

import jax
import jax.numpy as jnp
from jax.experimental import pallas as pl
from jax.experimental.pallas import tpu as pltpu

_K_HIGH = 16
_K_LOW = 2


def _block_body(sig_ref, mix_ref, scl_ref, x_ref, gain_ref, bias_ref,
                out_ref, sim_ref, *, bt: int, t: int, q0: int, klen: int):
    i = pl.program_id(1)
    xk = x_ref[0]
    q = x_ref[0, pl.ds((q0 + i) * bt, bt), :]

    kn = xk / jnp.clip(jnp.sqrt(jnp.sum(xk * xk, axis=1, keepdims=True)),
                       1e-12, None)
    qn = q / jnp.clip(jnp.sqrt(jnp.sum(q * q, axis=1, keepdims=True)),
                      1e-12, None)

    sim = jax.lax.dot_general(qn, kn, (((1,), (1,)), ((), ())),
                              preferred_element_type=jnp.float32)

    iota_s = jax.lax.broadcasted_iota(jnp.int32, (bt, klen), 1)
    t_glob = (q0 + i) * bt + jax.lax.broadcasted_iota(jnp.int32, (bt, klen), 0)
    past = iota_s < t_glob
    sim_ref[...] = jnp.where(past, sim, jnp.float32(-1e9))

    sigma = sig_ref[0, 0]
    surp = jnp.tanh(sigma * jnp.mean(jnp.abs(q), axis=1, keepdims=True))
    kt = jnp.clip(jnp.round(_K_LOW + (_K_HIGH - _K_LOW) * surp),
                  0.0, float(min(_K_HIGH, t - 1)))

    vals, idxs = [], []
    for j in range(_K_HIGH):
        s = sim_ref[...]
        cur = jnp.max(s, axis=1, keepdims=True)
        idx = jnp.argmax(s, axis=1).reshape(bt, 1)
        sim_ref[...] = jnp.where(iota_s == idx, jnp.float32(-2e9), s)
        vals.append(cur)
        idxs.append(idx)
    v16 = jnp.concatenate(vals, axis=1)
    i16 = jnp.concatenate(idxs, axis=1)

    jj = jax.lax.broadcasted_iota(jnp.int32, (bt, _K_HIGH), 1)
    kti = kt.astype(jnp.int32)
    sel = jnp.logical_and(jj < kti, v16 > -1e8)
    deg = jnp.maximum(jnp.sum(sel.astype(jnp.float32), axis=1,
                              keepdims=True), 1.0)
    isk = jj == (kti - 1)
    vstar = jnp.sum(jnp.where(isk, v16, 0.0), axis=1, keepdims=True)
    istar = jnp.max(jnp.where(isk, i16, -1), axis=1, keepdims=True)

    a = jnp.logical_and(
        jnp.logical_or(sim > vstar,
                       jnp.logical_and(sim == vstar, iota_s <= istar)),
        past).astype(jnp.float32)

    msg = jax.lax.dot_general(a, xk, (((1,), (0,)), ((), ())),
                              preferred_element_type=jnp.float32)
    msg = msg / deg

    mix = mix_ref[0, 0]
    scale = scl_ref[0, 0]
    blended = mix * q + (1.0 - mix) * msg
    y = blended * gain_ref[0] + bias_ref[0]
    gelu = 0.5 * y * (1.0 + jax.lax.erf(y * jnp.float32(0.7071067811865476)))
    out_ref[0] = gelu * scale


def _window_call(x, sigma, mix, scale, gain2d, bias2d, *, bt: int,
                 q0: int, nq: int, klen: int, interpret: bool = False):
    b, t, d = x.shape
    return pl.pallas_call(
        functools.partial(_block_body, bt=bt, t=t, q0=q0, klen=klen),
        grid=(b, nq),
        in_specs=[
            pl.BlockSpec((1, 1), lambda bb, ii: (0, 0),
                         memory_space=pltpu.SMEM),
            pl.BlockSpec((1, 1), lambda bb, ii: (0, 0),
                         memory_space=pltpu.SMEM),
            pl.BlockSpec((1, 1), lambda bb, ii: (0, 0),
                         memory_space=pltpu.SMEM),
            pl.BlockSpec((1, klen, d), lambda bb, ii: (bb, 0, 0)),
            pl.BlockSpec((1, d), lambda bb, ii: (0, 0)),
            pl.BlockSpec((1, d), lambda bb, ii: (0, 0)),
        ],
        out_specs=pl.BlockSpec((1, bt, d), lambda bb, ii: (bb, ii, 0)),
        out_shape=jax.ShapeDtypeStruct((b, nq * bt, d), jnp.float32),
        scratch_shapes=[pltpu.VMEM((bt, klen), jnp.float32)],
        interpret=interpret,
    )(sigma, mix, scale, x, gain2d, bias2d)


@functools.partial(jax.jit, static_argnames=("interpret",))
def kernel(x, gain, bias, log_sigma_raw, log_mix, log_scale,
           interpret: bool = False):
    b, t, d = x.shape
    bt = 256
    ni = t // bt

    sigma = (jax.nn.softplus(log_sigma_raw) + 0.01).reshape(1, 1)
    sigma = sigma.astype(jnp.float32)
    mix = jax.nn.sigmoid(log_mix).reshape(1, 1).astype(jnp.float32)
    scale = (jax.nn.softplus(log_scale) + 0.01).reshape(1, 1)
    scale = scale.astype(jnp.float32)
    gain2d = gain.reshape(1, d)
    bias2d = bias.reshape(1, d)

    pieces = []
    nq_step = 2
    for q0 in range(0, ni, nq_step):
        nq = min(nq_step, ni - q0)
        klen = (q0 + nq) * bt
        pieces.append(_window_call(x, sigma, mix, scale, gain2d, bias2d,
                                   bt=bt, q0=q0, nq=nq, klen=klen,
                                   interpret=interpret))
    return jnp.concatenate(pieces, axis=1)


# speedup vs baseline: 8.1159x; 1.2121x over previous
import functools

import jax
import jax.numpy as jnp
from jax.experimental import pallas as pl
from jax.experimental.pallas import tpu as pltpu

_K_HIGH = 16
_K_LOW = 2


def _block_body(sig_ref, mix_ref, scl_ref, x_ref, gain_ref, bias_ref,
                out_ref, sim_ref, *, bt: int, t: int, d: int):
    i = pl.program_id(1)
    xk = x_ref[0]
    q = x_ref[0, pl.ds(i * bt, bt), :]

    kn = xk / jnp.clip(jnp.sqrt(jnp.sum(xk * xk, axis=1, keepdims=True)),
                       1e-12, None)
    qn = q / jnp.clip(jnp.sqrt(jnp.sum(q * q, axis=1, keepdims=True)),
                      1e-12, None)

    sim = jax.lax.dot_general(qn, kn, (((1,), (1,)), ((), ())),
                              preferred_element_type=jnp.float32)

    iota_s = jax.lax.broadcasted_iota(jnp.int32, (bt, t), 1)
    t_glob = i * bt + jax.lax.broadcasted_iota(jnp.int32, (bt, t), 0)
    past = iota_s < t_glob
    s0 = jnp.where(past, sim, jnp.float32(-1e9))
    sim_ref[...] = s0

    sigma = sig_ref[0, 0]
    surp = jnp.tanh(sigma * jnp.mean(jnp.abs(q), axis=1, keepdims=True))
    kt = jnp.clip(jnp.round(_K_LOW + (_K_HIGH - _K_LOW) * surp),
                  0.0, float(min(_K_HIGH, t - 1)))
    kti = kt.astype(jnp.int32)

    idxs = []
    for j in range(_K_HIGH):
        s = sim_ref[...]
        idx = jnp.argmax(s, axis=1).reshape(bt, 1)
        sim_ref[...] = jnp.where(iota_s == idx, jnp.float32(-2e9), s)
        idxs.append(idx)
    i16 = jnp.concatenate(idxs, axis=1)

    jj = jax.lax.broadcasted_iota(jnp.int32, (bt, _K_HIGH), 1)
    isk = jj == (kti - 1)
    istar = jnp.max(jnp.where(isk, i16, -1), axis=1, keepdims=True)
    vstar = jnp.sum(jnp.where(iota_s == istar, s0, 0.0), axis=1,
                    keepdims=True)

    trow = (i * bt + jax.lax.broadcasted_iota(
        jnp.int32, (bt, 1), 0)).astype(jnp.float32)
    deg = jnp.maximum(jnp.minimum(kt, trow), 1.0)

    a = jnp.logical_and(
        jnp.logical_or(sim > vstar,
                       jnp.logical_and(sim == vstar, iota_s <= istar)),
        past).astype(jnp.float32)

    msg = jax.lax.dot_general(a, xk, (((1,), (0,)), ((), ())),
                              preferred_element_type=jnp.float32)
    msg = msg / deg

    mix = mix_ref[0, 0]
    scale = scl_ref[0, 0]
    blended = mix * q + (1.0 - mix) * msg
    y = blended * gain_ref[0] + bias_ref[0]
    gelu = 0.5 * y * (1.0 + jax.lax.erf(y * jnp.float32(0.7071067811865476)))
    out_ref[0] = gelu * scale


@functools.partial(jax.jit, static_argnames=("interpret", "bt"))
def kernel(x, gain, bias, log_sigma_raw, log_mix, log_scale,
           interpret: bool = False, bt: int = 512):
    b, t, d = x.shape
    bt = min(bt, t)
    ni = t // bt

    sigma = (jax.nn.softplus(log_sigma_raw) + 0.01).reshape(1, 1)
    mix = jax.nn.sigmoid(log_mix).reshape(1, 1)
    scale = (jax.nn.softplus(log_scale) + 0.01).reshape(1, 1)

    grid = (b, ni)
    out = pl.pallas_call(
        functools.partial(_block_body, bt=bt, t=t, d=d),
        grid=grid,
        in_specs=[
            pl.BlockSpec((1, 1), lambda bb, ii: (0, 0),
                         memory_space=pltpu.SMEM),
            pl.BlockSpec((1, 1), lambda bb, ii: (0, 0),
                         memory_space=pltpu.SMEM),
            pl.BlockSpec((1, 1), lambda bb, ii: (0, 0),
                         memory_space=pltpu.SMEM),
            pl.BlockSpec((1, t, d), lambda bb, ii: (bb, 0, 0)),
            pl.BlockSpec((1, d), lambda bb, ii: (0, 0)),
            pl.BlockSpec((1, d), lambda bb, ii: (0, 0)),
        ],
        out_specs=pl.BlockSpec((1, bt, d), lambda bb, ii: (bb, ii, 0)),
        out_shape=jax.ShapeDtypeStruct((b, t, d), jnp.float32),
        scratch_shapes=[
            pltpu.VMEM((bt, t), jnp.float32),
        ],
        interpret=interpret,
    )(sigma.astype(jnp.float32), mix.astype(jnp.float32),
      scale.astype(jnp.float32), x,
      gain.reshape(1, d), bias.reshape(1, d))
    return out
